# baseline (speedup 1.0000x reference)
import functools

import jax
import jax.numpy as jnp
from jax import lax
from jax.experimental import pallas as pl
from jax.experimental.pallas import tpu as pltpu
from jax.experimental.pallas import tpu_sc as plsc

VOCAB = 1000000
EMBED_DIM = 32
BATCH = 4096
SEQ = 200
IGNORE_INDEX = -100

B_TOTAL = BATCH * SEQ
NUM_WORKERS = 32
CHUNK = 1024
QB = BATCH // CHUNK
N_UNITS = SEQ * QB
UNITS_PER_W = N_UNITS // NUM_WORKERS
LANES = 16
TR = EMBED_DIM // 8
TCL = CHUNK // 128
JP = 129

_mesh = plsc.VectorSubcoreMesh(core_axis_name="c", subcore_axis_name="s")


@functools.partial(
    pl.kernel,
    mesh=_mesh,
    out_type=jax.ShapeDtypeStruct((SEQ, TR, BATCH // 128, 8, 128),
                                  jnp.float32),
    scratch_types=[
        pltpu.VMEM((CHUNK,), jnp.int32),
        pltpu.VMEM((CHUNK,), jnp.int32),
        pltpu.VMEM((CHUNK, EMBED_DIM), jnp.float32),
        pltpu.VMEM((CHUNK, EMBED_DIM), jnp.float32),
        pltpu.VMEM((TR, TCL, 8, JP), jnp.float32),
        pltpu.SemaphoreType.DMA,
        pltpu.SemaphoreType.DMA,
        pltpu.SemaphoreType.DMA,
        pltpu.SemaphoreType.DMA,
        pltpu.SemaphoreType.DMA,
    ],
    compiler_params=pltpu.CompilerParams(use_tc_tiling_on_sc=False,
                                         needs_layout_passes=False),
    cost_estimate=pl.CostEstimate(
        flops=0, transcendentals=0, bytes_accessed=213_000_000),
)
def _embed_sc(idx_hbm, table_hbm, out_hbm, idx0, idx1, rows0, rows1, st,
              si0, si1, sg0, sg1, so):
    wid = lax.axis_index("s") * 2 + lax.axis_index("c")
    u0 = wid * UNITS_PER_W
    idx_v = (idx0, idx1)
    rows_v = (rows0, rows1)
    sem_i = (si0, si1)
    sem_g = (sg0, sg1)

    l16 = lax.iota(jnp.int32, LANES)
    lo_tr = l16 // 8
    lo_j8 = l16 % 8
    hi_tr = lo_tr + 2
    ones = jnp.ones((LANES,), jnp.int32)

    def start_idx(k, b):
        pltpu.make_async_copy(idx_hbm.at[pl.ds((u0 + k) * CHUNK, CHUNK)],
                              idx_v[b], sem_i[b]).start()

    def wait_idx(b):
        pltpu.make_async_copy(idx_hbm.at[pl.ds(0, CHUNK)],
                              idx_v[b], sem_i[b]).wait()

    def fix(b):
        def fix_body(j, c):
            v = idx_v[b][pl.ds(j * LANES, LANES)]
            idx_v[b][pl.ds(j * LANES, LANES)] = jnp.where(
                v == IGNORE_INDEX, 1, v)
            return c
        lax.fori_loop(0, CHUNK // LANES, fix_body, 0)

    def start_gather(b):
        pltpu.make_async_copy(table_hbm.at[idx_v[b]],
                              rows_v[b], sem_g[b]).start()

    def wait_gather(b):
        pltpu.make_async_copy(table_hbm.at[idx_v[b]],
                              rows_v[b], sem_g[b]).wait()

    def transpose(b):
        rows = rows_v[b]

        @plsc.parallel_loop(0, TCL, 1)
        def _(tcl):
            tclv = ones * tcl

            @plsc.parallel_loop(0, 128, 1, unroll=4)
            def _(b7):
                i = tcl * 128 + b7
                b7v = ones * b7
                lo = rows[i, pl.ds(0, LANES)]
                hi = rows[i, pl.ds(LANES, LANES)]
                plsc.store_scatter(st, [lo_tr, tclv, lo_j8, b7v], lo)
                plsc.store_scatter(st, [hi_tr, tclv, lo_j8, b7v], hi)

    def start_out(k):
        u = u0 + k
        s = u // QB
        qb = u % QB
        for tr in range(TR):
            pltpu.make_async_copy(
                st.at[tr, :, :, pl.ds(0, 128)],
                out_hbm.at[s, tr, pl.ds(qb * TCL, TCL)], so).start()

    def wait_out():
        for tr in range(TR):
            pltpu.make_async_copy(
                st.at[tr, :, :, pl.ds(0, 128)],
                out_hbm.at[0, tr, pl.ds(0, TCL)], so).wait()

    start_idx(0, 0)
    wait_idx(0)
    fix(0)
    start_gather(0)
    start_idx(1, 1)

    def step(k, b):
        nb = 1 - b

        @pl.when(k + 1 < UNITS_PER_W)
        def _():
            wait_idx(nb)
            fix(nb)

        wait_gather(b)

        @pl.when(k + 1 < UNITS_PER_W)
        def _():
            start_gather(nb)

        @pl.when(k >= 1)
        def _():
            wait_out()

        transpose(b)
        start_out(k)

        @pl.when(k + 2 < UNITS_PER_W)
        def _():
            start_idx(k + 2, b)

    def pair(k2, c):
        for b in (0, 1):
            k = k2 * 2 + b

            @pl.when(k < UNITS_PER_W)
            def _():
                step(k, b)
        return c

    lax.fori_loop(0, (UNITS_PER_W + 1) // 2, pair, 0)
    wait_out()


_DETILE_V = 512
_DETILE_GRID = (VOCAB + _DETILE_V - 1) // _DETILE_V


def _detile_body(t_ref, out_ref):
    t = t_ref[...].T.reshape(128, 4, EMBED_DIM)
    for j in range(4):
        out_ref[:, j * EMBED_DIM:(j + 1) * EMBED_DIM] = t[:, j, :]


def _detile(table_t):
    return pl.pallas_call(
        _detile_body,
        grid=(_DETILE_GRID,),
        in_specs=[pl.BlockSpec((EMBED_DIM, _DETILE_V), lambda j: (0, j))],
        out_specs=pl.BlockSpec((128, 128), lambda j: (j, 0)),
        out_shape=jax.ShapeDtypeStruct((VOCAB * EMBED_DIM // 128, 128),
                                       jnp.float32),
    )(table_t)


def kernel(input, table):
    idx = input.T.reshape(-1)
    tbl_rm = _detile(table.T).reshape(VOCAB, EMBED_DIM)
    out6 = _embed_sc(idx, tbl_rm)
    out = out6.transpose(2, 4, 0, 1, 3).reshape(BATCH, SEQ, EMBED_DIM)
    return out


# speedup vs baseline: 2.9918x; 2.9918x over previous
import functools

import jax
import jax.numpy as jnp
from jax import lax
from jax.experimental import pallas as pl
from jax.experimental.pallas import tpu as pltpu
from jax.experimental.pallas import tpu_sc as plsc

VOCAB = 1000000
EMBED_DIM = 32
BATCH = 4096
SEQ = 200
IGNORE_INDEX = -100

B_TOTAL = BATCH * SEQ
NUM_WORKERS = 32
CHUNK = 1024
QB = BATCH // CHUNK
N_UNITS = SEQ * QB
UNITS_PER_W = N_UNITS // NUM_WORKERS
LANES = 16
TR = EMBED_DIM // 8
TCL = CHUNK // 128
JP = 129

_mesh = plsc.VectorSubcoreMesh(core_axis_name="c", subcore_axis_name="s")


@functools.partial(
    pl.kernel,
    mesh=_mesh,
    out_type=jax.ShapeDtypeStruct((SEQ, TR, BATCH // 128, 8, 128),
                                  jnp.float32),
    scratch_types=[
        pltpu.VMEM((CHUNK,), jnp.int32),
        pltpu.VMEM((CHUNK,), jnp.int32),
        pltpu.VMEM((CHUNK, EMBED_DIM), jnp.float32),
        pltpu.VMEM((CHUNK, EMBED_DIM), jnp.float32),
        pltpu.VMEM((TR, TCL, 8, JP), jnp.float32),
        pltpu.SemaphoreType.DMA,
        pltpu.SemaphoreType.DMA,
        pltpu.SemaphoreType.DMA,
        pltpu.SemaphoreType.DMA,
        pltpu.SemaphoreType.DMA,
    ],
    compiler_params=pltpu.CompilerParams(use_tc_tiling_on_sc=False,
                                         needs_layout_passes=False),
    cost_estimate=pl.CostEstimate(
        flops=0, transcendentals=0, bytes_accessed=213_000_000),
)
def _embed_sc(idx_hbm, table_hbm, out_hbm, idx0, idx1, rows0, rows1, st,
              si0, si1, sg0, sg1, so):
    wid = lax.axis_index("s") * 2 + lax.axis_index("c")
    u0 = wid * UNITS_PER_W
    idx_v = (idx0, idx1)
    rows_v = (rows0, rows1)
    sem_i = (si0, si1)
    sem_g = (sg0, sg1)

    l16 = lax.iota(jnp.int32, LANES)
    lo_tr = l16 // 8
    lo_j8 = l16 % 8
    hi_tr = lo_tr + 2
    ones = jnp.ones((LANES,), jnp.int32)

    def start_idx(k, b):
        pltpu.make_async_copy(idx_hbm.at[pl.ds((u0 + k) * CHUNK, CHUNK)],
                              idx_v[b], sem_i[b]).start()

    def wait_idx(b):
        pltpu.make_async_copy(idx_hbm.at[pl.ds(0, CHUNK)],
                              idx_v[b], sem_i[b]).wait()

    def fix(b):
        def fix_body(j, c):
            v = idx_v[b][pl.ds(j * LANES, LANES)]
            v = jnp.where(v == IGNORE_INDEX, 1, v)
            r = (v & -2048) + ((v & 511) << 2) + ((v >> 9) & 3)
            idx_v[b][pl.ds(j * LANES, LANES)] = r
            return c
        lax.fori_loop(0, CHUNK // LANES, fix_body, 0)

    def start_gather(b):
        pltpu.make_async_copy(table_hbm.at[idx_v[b]],
                              rows_v[b], sem_g[b]).start()

    def wait_gather(b):
        pltpu.make_async_copy(table_hbm.at[idx_v[b]],
                              rows_v[b], sem_g[b]).wait()

    def transpose(b):
        rows = rows_v[b]

        @plsc.parallel_loop(0, TCL, 1)
        def _(tcl):
            tclv = ones * tcl

            @plsc.parallel_loop(0, 128, 1, unroll=4)
            def _(b7):
                i = tcl * 128 + b7
                b7v = ones * b7
                lo = rows[i, pl.ds(0, LANES)]
                hi = rows[i, pl.ds(LANES, LANES)]
                plsc.store_scatter(st, [lo_tr, tclv, lo_j8, b7v], lo)
                plsc.store_scatter(st, [hi_tr, tclv, lo_j8, b7v], hi)

    def start_out(k):
        u = u0 + k
        s = u // QB
        qb = u % QB
        for tr in range(TR):
            pltpu.make_async_copy(
                st.at[tr, :, :, pl.ds(0, 128)],
                out_hbm.at[s, tr, pl.ds(qb * TCL, TCL)], so).start()

    def wait_out():
        for tr in range(TR):
            pltpu.make_async_copy(
                st.at[tr, :, :, pl.ds(0, 128)],
                out_hbm.at[0, tr, pl.ds(0, TCL)], so).wait()

    start_idx(0, 0)
    wait_idx(0)
    fix(0)
    start_gather(0)
    start_idx(1, 1)

    def step(k, b):
        nb = 1 - b

        @pl.when(k + 1 < UNITS_PER_W)
        def _():
            wait_idx(nb)
            fix(nb)

        wait_gather(b)

        @pl.when(k + 1 < UNITS_PER_W)
        def _():
            start_gather(nb)

        @pl.when(k >= 1)
        def _():
            wait_out()

        transpose(b)
        start_out(k)

        @pl.when(k + 2 < UNITS_PER_W)
        def _():
            start_idx(k + 2, b)

    def pair(k2, c):
        for b in (0, 1):
            k = k2 * 2 + b

            @pl.when(k < UNITS_PER_W)
            def _():
                step(k, b)
        return c

    lax.fori_loop(0, (UNITS_PER_W + 1) // 2, pair, 0)
    wait_out()


_DETILE_V = 2048
_DETILE_GRID = (VOCAB + _DETILE_V - 1) // _DETILE_V
_VSLOTS = _DETILE_GRID * _DETILE_V


def _detile_body(t_ref, out_ref):
    x = t_ref[...]
    z = jnp.concatenate([x[:, c * 512:(c + 1) * 512] for c in range(4)],
                        axis=0)
    out_ref[...] = z.T


def _detile(table_t):
    return pl.pallas_call(
        _detile_body,
        grid=(_DETILE_GRID,),
        in_specs=[pl.BlockSpec((EMBED_DIM, _DETILE_V), lambda j: (0, j))],
        out_specs=pl.BlockSpec((512, 128), lambda j: (j, 0)),
        out_shape=jax.ShapeDtypeStruct((_VSLOTS // 4, 128), jnp.float32),
    )(table_t)


def kernel(input, table):
    idx = input.T.reshape(-1)
    tbl_rm = _detile(table.T).reshape(_VSLOTS, EMBED_DIM)
    out6 = _embed_sc(idx, tbl_rm)
    out = out6.transpose(2, 4, 0, 1, 3).reshape(BATCH, SEQ, EMBED_DIM)
    return out
